# baseline (speedup 1.0000x reference)
import jax
import jax.numpy as jnp
from jax.experimental import pallas as pl

_K = 32
_BLOCK = 256


def _knn_body(ph_ref, plt_ref, x_ref, out_ref):
    ph = ph_ref[...]
    plt = plt_ref[...]
    hh = jnp.sum(ph * ph, axis=1, keepdims=True)
    ll = jnp.sum(plt * plt, axis=0, keepdims=True)
    cross = jax.lax.dot_general(
        ph, plt, (((1,), (0,)), ((), ())),
        preferred_element_type=jnp.float32)
    d2 = (hh + ll) - 2.0 * cross
    sq = jnp.maximum(d2, 0.0)
    key = jax.lax.bitcast_convert_type(sq, jnp.int32)

    nl = sq.shape[1]
    m = sq[:, 0:128]
    for c in range(1, nl // 128):
        m = jnp.minimum(m, sq[:, c * 128:(c + 1) * 128])
    lo0 = jax.lax.bitcast_convert_type(
        jnp.min(m, axis=1, keepdims=True), jnp.int32)
    hi0 = jax.lax.bitcast_convert_type(
        jnp.max(m, axis=1, keepdims=True), jnp.int32)

    def cond(carry):
        lo, hi = carry
        return jnp.any(lo < hi)

    def step(carry):
        lo, hi = carry
        mid = lo + (hi - lo) // 2
        cnt = jnp.sum((key <= mid).astype(jnp.int32), axis=1, keepdims=True)
        take = cnt >= _K
        lo = jnp.where(take, lo, mid + 1)
        hi = jnp.where(take, mid, hi)
        return lo, hi

    _, thresh = jax.lax.while_loop(cond, step, (lo0, hi0))

    w = jnp.where(key <= thresh, 1.0 / jnp.maximum(sq, 1e-16), 0.0)
    num = jax.lax.dot_general(
        w, x_ref[...], (((1,), (0,)), ((), ())),
        preferred_element_type=jnp.float32,
        precision=jax.lax.Precision.HIGHEST)
    den = jnp.sum(w, axis=1, keepdims=True)
    out_ref[...] = num / den


def kernel(x, pos_l, pos_h):
    nh = pos_h.shape[0]
    nl = pos_l.shape[0]
    d = x.shape[1]
    plt = pos_l.T
    return pl.pallas_call(
        _knn_body,
        grid=(nh // _BLOCK,),
        in_specs=[
            pl.BlockSpec((_BLOCK, 3), lambda i: (i, 0)),
            pl.BlockSpec((3, nl), lambda i: (0, 0)),
            pl.BlockSpec((nl, d), lambda i: (0, 0)),
        ],
        out_specs=pl.BlockSpec((_BLOCK, d), lambda i: (i, 0)),
        out_shape=jax.ShapeDtypeStruct((nh, d), x.dtype),
    )(pos_h, plt, x)


# speedup vs baseline: 1.2188x; 1.2188x over previous
import jax
import jax.numpy as jnp
from jax.experimental import pallas as pl

_K = 32
_BLOCK = 512


def _knn_body(ph_ref, plt_ref, x_ref, out_ref):
    ph = ph_ref[...]
    plt = plt_ref[...]
    hh = jnp.sum(ph * ph, axis=1, keepdims=True)
    ll = jnp.sum(plt * plt, axis=0, keepdims=True)
    cross = jax.lax.dot_general(
        ph, plt, (((1,), (0,)), ((), ())),
        preferred_element_type=jnp.float32)
    d2 = (hh + ll) - 2.0 * cross
    sq = jnp.maximum(d2, 0.0)
    key = jax.lax.bitcast_convert_type(sq, jnp.int32)

    b = ph.shape[0]

    def step(_, carry):
        lo, hi = carry
        mid = lo + (hi - lo) // 2
        cnt = jnp.sum((key <= mid).astype(jnp.int32), axis=1, keepdims=True)
        take = cnt >= _K
        lo = jnp.where(take, lo, mid + 1)
        hi = jnp.where(take, mid, hi)
        return lo, hi

    lo0 = jnp.zeros((b, 1), jnp.int32)
    hi0 = jnp.full((b, 1), 0x7F800000, jnp.int32)
    _, thresh = jax.lax.fori_loop(0, 31, step, (lo0, hi0))

    w = jnp.where(key <= thresh, 1.0 / jnp.maximum(sq, 1e-16), 0.0)
    x = x_ref[...]
    w_hi = w.astype(jnp.bfloat16)
    w_lo = (w - w_hi.astype(jnp.float32)).astype(jnp.bfloat16)
    x_hi = x.astype(jnp.bfloat16)
    x_lo = (x - x_hi.astype(jnp.float32)).astype(jnp.bfloat16)
    dims = (((1,), (0,)), ((), ()))
    num = (jax.lax.dot_general(w_hi, x_hi, dims,
                               preferred_element_type=jnp.float32)
           + jax.lax.dot_general(w_hi, x_lo, dims,
                                 preferred_element_type=jnp.float32)
           + jax.lax.dot_general(w_lo, x_hi, dims,
                                 preferred_element_type=jnp.float32))
    den = jnp.sum(w, axis=1, keepdims=True)
    out_ref[...] = num / den


def kernel(x, pos_l, pos_h):
    nh = pos_h.shape[0]
    nl = pos_l.shape[0]
    d = x.shape[1]
    plt = pos_l.T
    return pl.pallas_call(
        _knn_body,
        grid=(nh // _BLOCK,),
        in_specs=[
            pl.BlockSpec((_BLOCK, 3), lambda i: (i, 0)),
            pl.BlockSpec((3, nl), lambda i: (0, 0)),
            pl.BlockSpec((nl, d), lambda i: (0, 0)),
        ],
        out_specs=pl.BlockSpec((_BLOCK, d), lambda i: (i, 0)),
        out_shape=jax.ShapeDtypeStruct((nh, d), x.dtype),
    )(pos_h, plt, x)
